# baseline (speedup 1.0000x reference)
import functools

import jax
import jax.numpy as jnp
from jax import lax
from jax.experimental import pallas as pl
from jax.experimental.pallas import tpu as pltpu
from jax.experimental.pallas import tpu_sc as plsc

N = 10000
D = 128
E = 320000
K = 128
R1 = 160
R2 = 80
SB = 40
E_PAD = 16 * R1 * K
N_PAD = 10240
RPT = N_PAD // 16

_mesh = plsc.VectorSubcoreMesh(
    core_axis_name="c", subcore_axis_name="s", num_cores=2, num_subcores=16)


@functools.partial(
    pl.kernel,
    out_type=(
        jax.ShapeDtypeStruct((N_PAD, D), jnp.float32),
        jax.ShapeDtypeStruct((N_PAD, D), jnp.float32),
        jax.ShapeDtypeStruct((N_PAD,), jnp.float32),
        jax.ShapeDtypeStruct((N_PAD,), jnp.float32),
    ),
    mesh=_mesh,
    scratch_types=(
        pltpu.VMEM_SHARED((N_PAD, D), jnp.float32),
        pltpu.VMEM_SHARED((N_PAD,), jnp.float32),
        pltpu.VMEM((SB, K), jnp.int32),
        pltpu.VMEM((SB, K), jnp.int32),
        pltpu.VMEM((K, D), jnp.float32),
        pltpu.VMEM((K, D), jnp.float32),
        pltpu.VMEM((K,), jnp.float32),
        pltpu.SemaphoreType.DMA,
        pltpu.SemaphoreType.DMA,
    ),
)
def _sc_round1(src2d, dst2d, in_tbl, st_tbl, zeros2d, zeros1d,
               agg_in, agg_st, deg_a, deg_b,
               acc, dacc, src_v, dst_v, buf0, buf1, ones_v, sem0, sem1):
    cid = lax.axis_index("c")
    sid = lax.axis_index("s")
    zb = sid * RPT
    pltpu.sync_copy(zeros2d.at[pl.ds(zb, RPT)], acc.at[pl.ds(zb, RPT)])
    pltpu.sync_copy(zeros1d.at[pl.ds(zb, RPT)], dacc.at[pl.ds(zb, RPT)])

    for i in range(K // 16):
        ones_v[pl.ds(i * 16, 16)] = jnp.ones((16,), jnp.float32)

    base = sid * R1
    plsc.subcore_barrier()

    def blk_body(b, carry):
        pltpu.sync_copy(src2d.at[pl.ds(base + b * SB, SB)], src_v)
        pltpu.sync_copy(dst2d.at[pl.ds(base + b * SB, SB)], dst_v)

        def gather(j, buf, sem):
            @pl.when(cid == 0)
            def _():
                pltpu.async_copy(in_tbl.at[src_v.at[j]], buf, sem)

            @pl.when(cid == 1)
            def _():
                pltpu.async_copy(st_tbl.at[src_v.at[j]], buf, sem)

        def gwait(j, buf, sem):
            @pl.when(cid == 0)
            def _():
                pltpu.make_async_copy(in_tbl.at[src_v.at[j]], buf, sem).wait()

            @pl.when(cid == 1)
            def _():
                pltpu.make_async_copy(st_tbl.at[src_v.at[j]], buf, sem).wait()

        gather(0, buf0, sem0)

        def body(h, carry2):
            j0 = 2 * h
            j1 = j0 + 1
            gather(j1, buf1, sem1)
            gwait(j0, buf0, sem0)
            pltpu.sync_copy(buf0, acc.at[dst_v.at[j0]], add=True)

            @pl.when(h < SB // 2 - 1)
            def _():
                gather(j0 + 2, buf0, sem0)

            gwait(j1, buf1, sem1)
            pltpu.sync_copy(buf1, acc.at[dst_v.at[j1]], add=True)

            @pl.when(cid == 0)
            def _():
                pltpu.sync_copy(ones_v, dacc.at[dst_v.at[j0]], add=True)

            @pl.when(cid == 1)
            def _():
                pltpu.sync_copy(ones_v, dacc.at[dst_v.at[j1]], add=True)

            return carry2

        lax.fori_loop(0, SB // 2, body, 0)
        return carry

    lax.fori_loop(0, R1 // SB, blk_body, 0)
    plsc.subcore_barrier()

    @pl.when(cid == 0)
    def _():
        pltpu.sync_copy(acc.at[pl.ds(zb, RPT)], agg_in.at[pl.ds(zb, RPT)])
        pltpu.sync_copy(dacc.at[pl.ds(zb, RPT)], deg_a.at[pl.ds(zb, RPT)])

    @pl.when(cid == 1)
    def _():
        pltpu.sync_copy(acc.at[pl.ds(zb, RPT)], agg_st.at[pl.ds(zb, RPT)])
        pltpu.sync_copy(dacc.at[pl.ds(zb, RPT)], deg_b.at[pl.ds(zb, RPT)])


@functools.partial(
    pl.kernel,
    out_type=(
        jax.ShapeDtypeStruct((N_PAD, D), jnp.float32),
        jax.ShapeDtypeStruct((N_PAD, D), jnp.float32),
    ),
    mesh=_mesh,
    scratch_types=(
        pltpu.VMEM_SHARED((N_PAD, D), jnp.float32),
        pltpu.VMEM((SB, K), jnp.int32),
        pltpu.VMEM((SB, K), jnp.int32),
        pltpu.VMEM((K, D), jnp.float32),
        pltpu.VMEM((K, D), jnp.float32),
        pltpu.SemaphoreType.DMA,
        pltpu.SemaphoreType.DMA,
    ),
)
def _sc_round2(src2d, dst2d, rs_tbl, zeros2d,
               agg_a, agg_b,
               acc, src_v, dst_v, buf0, buf1, sem0, sem1):
    cid = lax.axis_index("c")
    sid = lax.axis_index("s")
    zb = sid * RPT
    pltpu.sync_copy(zeros2d.at[pl.ds(zb, RPT)], acc.at[pl.ds(zb, RPT)])

    base = cid * (16 * R2) + sid * R2
    plsc.subcore_barrier()

    def blk_body(b, carry):
        pltpu.sync_copy(src2d.at[pl.ds(base + b * SB, SB)], src_v)
        pltpu.sync_copy(dst2d.at[pl.ds(base + b * SB, SB)], dst_v)

        def gather(j, buf, sem):
            pltpu.async_copy(rs_tbl.at[src_v.at[j]], buf, sem)

        def gwait(j, buf, sem):
            pltpu.make_async_copy(rs_tbl.at[src_v.at[j]], buf, sem).wait()

        gather(0, buf0, sem0)

        def body(h, carry2):
            j0 = 2 * h
            j1 = j0 + 1
            gather(j1, buf1, sem1)
            gwait(j0, buf0, sem0)
            pltpu.sync_copy(buf0, acc.at[dst_v.at[j0]], add=True)

            @pl.when(h < SB // 2 - 1)
            def _():
                gather(j0 + 2, buf0, sem0)

            gwait(j1, buf1, sem1)
            pltpu.sync_copy(buf1, acc.at[dst_v.at[j1]], add=True)
            return carry2

        lax.fori_loop(0, SB // 2, body, 0)
        return carry

    lax.fori_loop(0, R2 // SB, blk_body, 0)
    plsc.subcore_barrier()

    @pl.when(cid == 0)
    def _():
        pltpu.sync_copy(acc.at[pl.ds(zb, RPT)], agg_a.at[pl.ds(zb, RPT)])

    @pl.when(cid == 1)
    def _():
        pltpu.sync_copy(acc.at[pl.ds(zb, RPT)], agg_b.at[pl.ds(zb, RPT)])


BLK = 1000


def _tc_pre(x_in, x_st, wg_s, wc_top, bias_g, pre_gr_ref, pre_gu_ref,
            pre_ct_ref):
    x = jnp.concatenate([x_in[...], x_st[...]], axis=1)
    pre_g = (jnp.dot(x, wg_s[...], preferred_element_type=jnp.float32)
             + bias_g[...])
    pre_gr_ref[...] = pre_g[:, :D]
    pre_gu_ref[...] = pre_g[:, D:]
    pre_ct_ref[...] = jnp.dot(x_in[...], wc_top[...],
                              preferred_element_type=jnp.float32)


def _tc_gate(x_st, agg_in, agg_st, deg_a, deg_b, pre_gr, pre_ct, wg_nr,
             wc_bot, wcn_top, bias_c, rs_ref, m_ref, pre_c_ref):
    inv = 1.0 / jnp.maximum(deg_a[...] + deg_b[...], 1.0)
    mi = agg_in[...] * inv
    ms = agg_st[...] * inv
    m = jnp.concatenate([mi, ms], axis=1)
    r = jax.nn.sigmoid(pre_gr[...] + jnp.dot(m, wg_nr[...],
                                             preferred_element_type=jnp.float32))
    rs = r * x_st[...]
    rs_ref[...] = rs
    m_ref[...] = m
    pre_c_ref[...] = (pre_ct[...]
                      + jnp.dot(rs, wc_bot[...],
                                preferred_element_type=jnp.float32)
                      + jnp.dot(mi, wcn_top[...],
                                preferred_element_type=jnp.float32)
                      + bias_c[...])


def _tc_u(m, pre_gu, wg_nu, u_ref):
    u_ref[...] = jax.nn.sigmoid(pre_gu[...] + jnp.dot(
        m[...], wg_nu[...], preferred_element_type=jnp.float32))


def _tc_out(x_st, u, pre_c, agg_a, agg_b, deg_a, deg_b, wcn_bot, out_ref):
    inv = 1.0 / jnp.maximum(deg_a[...] + deg_b[...], 1.0)
    mrs = (agg_a[...] + agg_b[...]) * inv
    c = jnp.tanh(pre_c[...] + jnp.dot(mrs, wcn_bot[...],
                                      preferred_element_type=jnp.float32))
    uu = u[...]
    out_ref[...] = uu * x_st[...] + (1.0 - uu) * c


def _row(i):
    return (i, 0)


def _full(i):
    return (0, 0)


def _pre_call(x_in, x_st, wg_s, wc_top, bias_g):
    return pl.pallas_call(
        _tc_pre,
        grid=(N // BLK,),
        in_specs=[
            pl.BlockSpec((BLK, D), _row),
            pl.BlockSpec((BLK, D), _row),
            pl.BlockSpec((2 * D, 2 * D), _full),
            pl.BlockSpec((D, D), _full),
            pl.BlockSpec((1, 2 * D), _full),
        ],
        out_specs=[pl.BlockSpec((BLK, D), _row)] * 3,
        out_shape=[jax.ShapeDtypeStruct((N, D), jnp.float32)] * 3,
    )(x_in, x_st, wg_s, wc_top, bias_g)


def _gate_call(x_st, agg_in, agg_st, deg_a, deg_b, pre_gr, pre_ct, wg_nr,
               wc_bot, wcn_top, bias_c):
    return pl.pallas_call(
        _tc_gate,
        grid=(N // BLK,),
        in_specs=[
            pl.BlockSpec((BLK, D), _row),
            pl.BlockSpec((BLK, D), _row),
            pl.BlockSpec((BLK, D), _row),
            pl.BlockSpec((BLK, 1), _row),
            pl.BlockSpec((BLK, 1), _row),
            pl.BlockSpec((BLK, D), _row),
            pl.BlockSpec((BLK, D), _row),
            pl.BlockSpec((2 * D, D), _full),
            pl.BlockSpec((D, D), _full),
            pl.BlockSpec((D, D), _full),
            pl.BlockSpec((1, D), _full),
        ],
        out_specs=[
            pl.BlockSpec((BLK, D), _row),
            pl.BlockSpec((BLK, 2 * D), _row),
            pl.BlockSpec((BLK, D), _row),
        ],
        out_shape=[
            jax.ShapeDtypeStruct((N, D), jnp.float32),
            jax.ShapeDtypeStruct((N, 2 * D), jnp.float32),
            jax.ShapeDtypeStruct((N, D), jnp.float32),
        ],
    )(x_st, agg_in, agg_st, deg_a, deg_b, pre_gr, pre_ct, wg_nr, wc_bot,
      wcn_top, bias_c)


def _u_call(m, pre_gu, wg_nu):
    return pl.pallas_call(
        _tc_u,
        grid=(N // BLK,),
        in_specs=[
            pl.BlockSpec((BLK, 2 * D), _row),
            pl.BlockSpec((BLK, D), _row),
            pl.BlockSpec((2 * D, D), _full),
        ],
        out_specs=pl.BlockSpec((BLK, D), _row),
        out_shape=jax.ShapeDtypeStruct((N, D), jnp.float32),
    )(m, pre_gu, wg_nu)


def _out_call(x_st, u, pre_c, agg_a, agg_b, deg_a, deg_b, wcn_bot):
    return pl.pallas_call(
        _tc_out,
        grid=(N // BLK,),
        in_specs=[
            pl.BlockSpec((BLK, D), _row),
            pl.BlockSpec((BLK, D), _row),
            pl.BlockSpec((BLK, D), _row),
            pl.BlockSpec((BLK, D), _row),
            pl.BlockSpec((BLK, D), _row),
            pl.BlockSpec((BLK, 1), _row),
            pl.BlockSpec((BLK, 1), _row),
            pl.BlockSpec((D, D), _full),
        ],
        out_specs=pl.BlockSpec((BLK, D), _row),
        out_shape=jax.ShapeDtypeStruct((N, D), jnp.float32),
    )(x_st, u, pre_c, agg_a, agg_b, deg_a, deg_b, wcn_bot)


def kernel(inputs, state, edge_index, output_nodes, Wg_self, Wg_neigh, bg,
           Wc_self, Wc_neigh, bc, gate_bias, candidate_bias):
    src = edge_index[0].astype(jnp.int32)
    dst = edge_index[1].astype(jnp.int32)
    pad = E_PAD - E
    src_pad = jnp.arange(pad, dtype=jnp.int32) % N
    dst_pad = N + (jnp.arange(pad, dtype=jnp.int32) % (N_PAD - N))
    src2d = jnp.concatenate([src, src_pad]).reshape(E_PAD // K, K)
    dst2d = jnp.concatenate([dst, dst_pad]).reshape(E_PAD // K, K)
    zeros2d = jnp.zeros((N_PAD, D), jnp.float32)
    zeros1d = jnp.zeros((N_PAD,), jnp.float32)

    bias_g = (bg + gate_bias).reshape(1, 2 * D)
    bias_c = (bc + candidate_bias).reshape(1, D)
    wc_top, wc_bot = Wc_self[:D], Wc_self[D:]
    wcn_top, wcn_bot = Wc_neigh[:D], Wc_neigh[D:]

    wg_nr, wg_nu = Wg_neigh[:, :D], Wg_neigh[:, D:]

    pre_gr, pre_gu, pre_ct = _pre_call(inputs, state, Wg_self, wc_top, bias_g)

    agg_in, agg_st, deg_a, deg_b = _sc_round1(src2d, dst2d, inputs, state,
                                              zeros2d, zeros1d)
    dega2d = deg_a.reshape(N_PAD, 1)
    degb2d = deg_b.reshape(N_PAD, 1)

    rs, m, pre_c = _gate_call(state, agg_in, agg_st, dega2d, degb2d, pre_gr,
                              pre_ct, wg_nr, wc_bot, wcn_top, bias_c)

    agg_a, agg_b = _sc_round2(src2d, dst2d, rs, zeros2d)
    u = _u_call(m, pre_gu, wg_nu)

    return _out_call(state, u, pre_c, agg_a, agg_b, dega2d, degb2d, wcn_bot)


# speedup vs baseline: 1.0117x; 1.0117x over previous
import functools

import jax
import jax.numpy as jnp
from jax import lax
from jax.experimental import pallas as pl
from jax.experimental.pallas import tpu as pltpu
from jax.experimental.pallas import tpu_sc as plsc

N = 10000
D = 128
E = 320000
K = 128
R1 = 160
R2 = 80
SB = 40
E_PAD = 16 * R1 * K
N_PAD = 10240
RPT = N_PAD // 16

_mesh = plsc.VectorSubcoreMesh(
    core_axis_name="c", subcore_axis_name="s", num_cores=2, num_subcores=16)


@functools.partial(
    pl.kernel,
    out_type=(
        jax.ShapeDtypeStruct((N_PAD, D), jnp.float32),
        jax.ShapeDtypeStruct((N_PAD, D), jnp.float32),
        jax.ShapeDtypeStruct((N_PAD,), jnp.float32),
        jax.ShapeDtypeStruct((N_PAD,), jnp.float32),
    ),
    mesh=_mesh,
    scratch_types=(
        pltpu.VMEM_SHARED((N_PAD, D), jnp.float32),
        pltpu.VMEM_SHARED((N_PAD,), jnp.float32),
        pltpu.VMEM((SB, K), jnp.int32),
        pltpu.VMEM((SB, K), jnp.int32),
        pltpu.VMEM((K, D), jnp.float32),
        pltpu.VMEM((K, D), jnp.float32),
        pltpu.VMEM((K,), jnp.float32),
        pltpu.SemaphoreType.DMA,
        pltpu.SemaphoreType.DMA,
    ),
)
def _sc_round1(src2d, dst2d, in_tbl, st_tbl, zeros2d, zeros1d,
               agg_in, agg_st, deg_a, deg_b,
               acc, dacc, src_v, dst_v, buf0, buf1, ones_v, sem0, sem1):
    cid = lax.axis_index("c")
    sid = lax.axis_index("s")
    zb = sid * RPT
    pltpu.sync_copy(zeros2d.at[pl.ds(zb, RPT)], acc.at[pl.ds(zb, RPT)])
    pltpu.sync_copy(zeros1d.at[pl.ds(zb, RPT)], dacc.at[pl.ds(zb, RPT)])

    for i in range(K // 16):
        ones_v[pl.ds(i * 16, 16)] = jnp.ones((16,), jnp.float32)

    base = sid * R1
    plsc.subcore_barrier()

    def blk_body(b, carry):
        pltpu.sync_copy(src2d.at[pl.ds(base + b * SB, SB)], src_v)
        pltpu.sync_copy(dst2d.at[pl.ds(base + b * SB, SB)], dst_v)

        def gather(j, buf, sem):
            @pl.when(cid == 0)
            def _():
                pltpu.async_copy(in_tbl.at[src_v.at[j]], buf, sem)

            @pl.when(cid == 1)
            def _():
                pltpu.async_copy(st_tbl.at[src_v.at[j]], buf, sem)

        def gwait(j, buf, sem):
            @pl.when(cid == 0)
            def _():
                pltpu.make_async_copy(in_tbl.at[src_v.at[j]], buf, sem).wait()

            @pl.when(cid == 1)
            def _():
                pltpu.make_async_copy(st_tbl.at[src_v.at[j]], buf, sem).wait()

        gather(0, buf0, sem0)

        def body(h, carry2):
            j0 = 2 * h
            j1 = j0 + 1
            gather(j1, buf1, sem1)
            gwait(j0, buf0, sem0)
            pltpu.sync_copy(buf0, acc.at[dst_v.at[j0]], add=True)

            @pl.when(h < SB // 2 - 1)
            def _():
                gather(j0 + 2, buf0, sem0)

            gwait(j1, buf1, sem1)
            pltpu.sync_copy(buf1, acc.at[dst_v.at[j1]], add=True)

            @pl.when(cid == 0)
            def _():
                pltpu.sync_copy(ones_v, dacc.at[dst_v.at[j0]], add=True)

            @pl.when(cid == 1)
            def _():
                pltpu.sync_copy(ones_v, dacc.at[dst_v.at[j1]], add=True)

            return carry2

        lax.fori_loop(0, SB // 2, body, 0)
        return carry

    lax.fori_loop(0, R1 // SB, blk_body, 0)
    plsc.subcore_barrier()

    @pl.when(cid == 0)
    def _():
        pltpu.sync_copy(acc.at[pl.ds(zb, RPT)], agg_in.at[pl.ds(zb, RPT)])
        pltpu.sync_copy(dacc.at[pl.ds(zb, RPT)], deg_a.at[pl.ds(zb, RPT)])

    @pl.when(cid == 1)
    def _():
        pltpu.sync_copy(acc.at[pl.ds(zb, RPT)], agg_st.at[pl.ds(zb, RPT)])
        pltpu.sync_copy(dacc.at[pl.ds(zb, RPT)], deg_b.at[pl.ds(zb, RPT)])


@functools.partial(
    pl.kernel,
    out_type=(
        jax.ShapeDtypeStruct((N_PAD, D), jnp.float32),
        jax.ShapeDtypeStruct((N_PAD, D), jnp.float32),
    ),
    mesh=_mesh,
    scratch_types=(
        pltpu.VMEM_SHARED((N_PAD, D), jnp.float32),
        pltpu.VMEM((SB, K), jnp.int32),
        pltpu.VMEM((SB, K), jnp.int32),
        pltpu.VMEM((K, D), jnp.float32),
        pltpu.VMEM((K, D), jnp.float32),
        pltpu.SemaphoreType.DMA,
        pltpu.SemaphoreType.DMA,
    ),
)
def _sc_round2(src2d, dst2d, rs_tbl, zeros2d,
               agg_a, agg_b,
               acc, src_v, dst_v, buf0, buf1, sem0, sem1):
    cid = lax.axis_index("c")
    sid = lax.axis_index("s")
    zb = sid * RPT
    pltpu.sync_copy(zeros2d.at[pl.ds(zb, RPT)], acc.at[pl.ds(zb, RPT)])

    base = cid * (16 * R2) + sid * R2
    plsc.subcore_barrier()

    def blk_body(b, carry):
        pltpu.sync_copy(src2d.at[pl.ds(base + b * SB, SB)], src_v)
        pltpu.sync_copy(dst2d.at[pl.ds(base + b * SB, SB)], dst_v)

        def gather(j, buf, sem):
            pltpu.async_copy(rs_tbl.at[src_v.at[j]], buf, sem)

        def gwait(j, buf, sem):
            pltpu.make_async_copy(rs_tbl.at[src_v.at[j]], buf, sem).wait()

        gather(0, buf0, sem0)

        def body(h, carry2):
            j0 = 2 * h
            j1 = j0 + 1
            gather(j1, buf1, sem1)
            gwait(j0, buf0, sem0)
            pltpu.sync_copy(buf0, acc.at[dst_v.at[j0]], add=True)

            @pl.when(h < SB // 2 - 1)
            def _():
                gather(j0 + 2, buf0, sem0)

            gwait(j1, buf1, sem1)
            pltpu.sync_copy(buf1, acc.at[dst_v.at[j1]], add=True)
            return carry2

        lax.fori_loop(0, SB // 2, body, 0)
        return carry

    lax.fori_loop(0, R2 // SB, blk_body, 0)
    plsc.subcore_barrier()

    @pl.when(cid == 0)
    def _():
        pltpu.sync_copy(acc.at[pl.ds(zb, RPT)], agg_a.at[pl.ds(zb, RPT)])

    @pl.when(cid == 1)
    def _():
        pltpu.sync_copy(acc.at[pl.ds(zb, RPT)], agg_b.at[pl.ds(zb, RPT)])


BLK = 1000


def _tc_gate(x_in, x_st, agg_in, agg_st, deg_a, deg_b, wg_s, wg_n, bias,
             u_ref, rs_ref, mi_ref):
    inv = 1.0 / jnp.maximum(deg_a[...] + deg_b[...], 1.0)
    mi = agg_in[...] * inv
    ms = agg_st[...] * inv
    x = jnp.concatenate([x_in[...], x_st[...]], axis=1)
    m = jnp.concatenate([mi, ms], axis=1)
    pre = (jnp.dot(x, wg_s[...], preferred_element_type=jnp.float32)
           + jnp.dot(m, wg_n[...], preferred_element_type=jnp.float32)
           + bias[...])
    g = jax.nn.sigmoid(pre)
    u_ref[...] = g[:, D:]
    rs_ref[...] = g[:, :D] * x_st[...]
    mi_ref[...] = mi


def _tc_out(x_in, x_st, rs, u, mi, agg_a, agg_b, deg_a, deg_b, wc_s, wc_n,
            bias, out_ref):
    inv = 1.0 / jnp.maximum(deg_a[...] + deg_b[...], 1.0)
    mrs = (agg_a[...] + agg_b[...]) * inv
    x2 = jnp.concatenate([x_in[...], rs[...]], axis=1)
    m2 = jnp.concatenate([mi[...], mrs], axis=1)
    c = jnp.tanh(jnp.dot(x2, wc_s[...], preferred_element_type=jnp.float32)
                 + jnp.dot(m2, wc_n[...], preferred_element_type=jnp.float32)
                 + bias[...])
    uu = u[...]
    out_ref[...] = uu * x_st[...] + (1.0 - uu) * c


def _row(i):
    return (i, 0)


def _full(i):
    return (0, 0)


def _gate_call(x_in, x_st, agg_in, agg_st, deg_a, deg_b, wg_s, wg_n, bias):
    return pl.pallas_call(
        _tc_gate,
        grid=(N // BLK,),
        in_specs=[
            pl.BlockSpec((BLK, D), _row),
            pl.BlockSpec((BLK, D), _row),
            pl.BlockSpec((BLK, D), _row),
            pl.BlockSpec((BLK, D), _row),
            pl.BlockSpec((BLK, 1), _row),
            pl.BlockSpec((BLK, 1), _row),
            pl.BlockSpec((2 * D, 2 * D), _full),
            pl.BlockSpec((2 * D, 2 * D), _full),
            pl.BlockSpec((1, 2 * D), _full),
        ],
        out_specs=[pl.BlockSpec((BLK, D), _row)] * 3,
        out_shape=[jax.ShapeDtypeStruct((N, D), jnp.float32)] * 3,
    )(x_in, x_st, agg_in, agg_st, deg_a, deg_b, wg_s, wg_n, bias)


def _out_call(x_in, x_st, rs, u, mi, agg_a, agg_b, deg_a, deg_b, wc_s, wc_n,
              bias):
    return pl.pallas_call(
        _tc_out,
        grid=(N // BLK,),
        in_specs=[
            pl.BlockSpec((BLK, D), _row),
            pl.BlockSpec((BLK, D), _row),
            pl.BlockSpec((BLK, D), _row),
            pl.BlockSpec((BLK, D), _row),
            pl.BlockSpec((BLK, D), _row),
            pl.BlockSpec((BLK, D), _row),
            pl.BlockSpec((BLK, D), _row),
            pl.BlockSpec((BLK, 1), _row),
            pl.BlockSpec((BLK, 1), _row),
            pl.BlockSpec((2 * D, D), _full),
            pl.BlockSpec((2 * D, D), _full),
            pl.BlockSpec((1, D), _full),
        ],
        out_specs=pl.BlockSpec((BLK, D), _row),
        out_shape=jax.ShapeDtypeStruct((N, D), jnp.float32),
    )(x_in, x_st, rs, u, mi, agg_a, agg_b, deg_a, deg_b, wc_s, wc_n, bias)


def kernel(inputs, state, edge_index, output_nodes, Wg_self, Wg_neigh, bg,
           Wc_self, Wc_neigh, bc, gate_bias, candidate_bias):
    src = edge_index[0].astype(jnp.int32)
    dst = edge_index[1].astype(jnp.int32)
    pad = E_PAD - E
    src_pad = jnp.arange(pad, dtype=jnp.int32) % N
    dst_pad = N + (jnp.arange(pad, dtype=jnp.int32) % (N_PAD - N))
    src2d = jnp.concatenate([src, src_pad]).reshape(E_PAD // K, K)
    dst2d = jnp.concatenate([dst, dst_pad]).reshape(E_PAD // K, K)
    zeros2d = jnp.zeros((N_PAD, D), jnp.float32)
    zeros1d = jnp.zeros((N_PAD,), jnp.float32)

    agg_in, agg_st, deg_a, deg_b = _sc_round1(src2d, dst2d, inputs, state,
                                              zeros2d, zeros1d)
    dega2d = deg_a.reshape(N_PAD, 1)
    degb2d = deg_b.reshape(N_PAD, 1)

    bias_g = (bg + gate_bias).reshape(1, 2 * D)
    u, rs, mi = _gate_call(inputs, state, agg_in, agg_st, dega2d, degb2d,
                           Wg_self, Wg_neigh, bias_g)

    agg_a, agg_b = _sc_round2(src2d, dst2d, rs, zeros2d)

    bias_c = (bc + candidate_bias).reshape(1, D)
    return _out_call(inputs, state, rs, u, mi, agg_a, agg_b, dega2d, degb2d,
                     Wc_self, Wc_neigh, bias_c)
